# baseline (speedup 1.0000x reference)
import jax
import jax.numpy as jnp
from jax import lax
from jax.experimental import pallas as pl
from jax.experimental.pallas import tpu as pltpu
from jax.experimental.pallas import tpu_sc as plsc

NUM_CL = 1000
NI = 4096
NJ = 50
NW = 32
KT = NUM_CL // 8
TPW = NJ * KT
NT = 25
CHUNK_W = NT * 1024
CHUNKS = TPW // NT
CPJ = KT // NT
NRING = 4


def _body(x_hbm, zeros_hbm, out_hbm, idx_v, pos_all, *scratch):
    bufs = scratch[:NRING]
    sems = scratch[NRING:]
    wid = lax.axis_index("c") * 16 + lax.axis_index("s")

    pltpu.sync_copy(x_hbm.at[pl.ds(wid * 128 * NJ, 128 * NJ)], idx_v)
    for s in range(NRING):
        pltpu.sync_copy(zeros_hbm, bufs[s])

    iota = lax.iota(jnp.int32, 16)
    ones_v = jnp.full((16,), 1.0, jnp.float32)
    zeros_v = jnp.zeros((16,), jnp.float32)

    def pos_body(j, carry):
        for v in range(8):
            i_loc = iota + 16 * v
            xv = plsc.load_gather(idx_v, [i_loc * NJ + j])
            pcol = ((xv >> 3) << 10) + ((xv & 7) << 7) + i_loc
            pos_all[j, pl.ds(16 * v, 16)] = pcol
        return carry

    lax.fori_loop(0, NJ, pos_body, jnp.int32(0), unroll=False)

    def put(c, s, val):
        j = c // CPJ
        lo = (c - j * CPJ) * CHUNK_W
        for v in range(8):
            pcol = pos_all[j, pl.ds(16 * v, 16)]
            rel = pcol - lo
            m = (rel >= 0) & (rel < CHUNK_W)
            plsc.store_scatter(bufs[s], [rel >> 10, (rel >> 7) & 7, rel & 127], val, mask=m)

    def fire(c, s):
        put(c, s, ones_v)
        dst = out_hbm.at[pl.ds(NT * c, NT), wid]
        pltpu.async_copy(bufs[s], dst, sems[s])

    def wait_slot(s):
        dst = out_hbm.at[pl.ds(0, NT), wid]
        pltpu.make_async_copy(bufs[s], dst, sems[s]).wait()

    for s in range(NRING):
        fire(jnp.int32(s), s)

    def round_body(g, carry):
        for s in range(NRING):
            c = g * NRING + s
            wait_slot(s)
            put(c - NRING, s, zeros_v)
            fire(c, s)
        return carry

    full_rounds = CHUNKS // NRING
    lax.fori_loop(1, full_rounds, round_body, jnp.int32(0), unroll=False)

    for c_tail in range(full_rounds * NRING, CHUNKS):
        s = c_tail % NRING
        wait_slot(s)
        put(jnp.int32(c_tail - NRING), s, zeros_v)
        fire(jnp.int32(c_tail), s)

    for s in range(NRING):
        wait_slot(s)


@jax.jit
def _onehot_sc(x_flat, zeros_tile):
    mesh = plsc.VectorSubcoreMesh(core_axis_name="c", subcore_axis_name="s")
    kern = pl.kernel(
        _body,
        out_type=jax.ShapeDtypeStruct((TPW, NW, 8, 128), jnp.float32),
        mesh=mesh,
        compiler_params=pltpu.CompilerParams(needs_layout_passes=False),
        scratch_types=(
            [pltpu.VMEM((128 * NJ,), jnp.int32),
             pltpu.VMEM((NJ, 128), jnp.int32)]
            + [pltpu.VMEM((NT, 8, 128), jnp.float32) for _ in range(NRING)]
            + [pltpu.SemaphoreType.DMA for _ in range(NRING)]
        ),
    )
    return kern(x_flat, zeros_tile)


def kernel(x):
    x_flat = x.reshape(NI * NJ).astype(jnp.int32)
    zeros_tile = jnp.zeros((NT, 8, 128), jnp.float32)
    out = _onehot_sc(x_flat, zeros_tile)
    o5 = out.reshape(NJ, KT, NW, 8, 128)
    return o5.transpose(2, 4, 0, 1, 3).reshape(NI, NJ, NUM_CL)


# speedup vs baseline: 1.0377x; 1.0377x over previous
import jax
import jax.numpy as jnp
from jax import lax
from jax.experimental import pallas as pl
from jax.experimental.pallas import tpu as pltpu
from jax.experimental.pallas import tpu_sc as plsc

NUM_CL = 1000
NI = 4096
NJ = 50
NW = 32
KT = NUM_CL // 8
TPW = NJ * KT
NT = 25
CHUNK_W = NT * 1024
CHUNKS = TPW // NT
CPJ = KT // NT
NRING = 2


def _body(x_hbm, zeros_hbm, out_hbm, idx_v, pos_all, *scratch):
    bufs = scratch[:NRING]
    sems = scratch[NRING:]
    wid = lax.axis_index("c") * 16 + lax.axis_index("s")

    for s in range(NRING):
        pltpu.async_copy(zeros_hbm, bufs[s], sems[s])
    pltpu.sync_copy(x_hbm.at[pl.ds(wid * 128 * NJ, 128 * NJ)], idx_v)

    iota = lax.iota(jnp.int32, 16)
    ones_v = jnp.full((16,), 1.0, jnp.float32)
    zeros_v = jnp.zeros((16,), jnp.float32)

    def pos_body(j, carry):
        for v in range(8):
            i_loc = iota + 16 * v
            xv = plsc.load_gather(idx_v, [i_loc * NJ + j])
            pcol = ((xv >> 3) << 10) + ((xv & 7) << 7) + i_loc
            pos_all[j, pl.ds(16 * v, 16)] = pcol
        return carry

    pos_body(jnp.int32(0), jnp.int32(0))
    for s in range(NRING):
        pltpu.make_async_copy(zeros_hbm, bufs[s], sems[s]).wait()

    def put(c, s, val):
        j = c // CPJ
        lo = (c - j * CPJ) * CHUNK_W
        for v in range(8):
            pcol = pos_all[j, pl.ds(16 * v, 16)]
            rel = pcol - lo
            m = (rel >= 0) & (rel < CHUNK_W)
            plsc.store_scatter(bufs[s], [rel >> 10, (rel >> 7) & 7, rel & 127], val, mask=m)

    def fire(c, s):
        put(c, s, ones_v)
        dst = out_hbm.at[pl.ds(NT * c, NT), wid]
        pltpu.async_copy(bufs[s], dst, sems[s])

    def wait_slot(s):
        dst = out_hbm.at[pl.ds(0, NT), wid]
        pltpu.make_async_copy(bufs[s], dst, sems[s]).wait()

    for s in range(NRING):
        fire(jnp.int32(s), s)
    lax.fori_loop(1, NJ, pos_body, jnp.int32(0), unroll=False)

    def round_body(g, carry):
        for s in range(NRING):
            c = g * NRING + s
            wait_slot(s)
            put(c - NRING, s, zeros_v)
            fire(c, s)
        return carry

    full_rounds = CHUNKS // NRING
    lax.fori_loop(1, full_rounds, round_body, jnp.int32(0), unroll=False)

    for c_tail in range(full_rounds * NRING, CHUNKS):
        s = c_tail % NRING
        wait_slot(s)
        put(jnp.int32(c_tail - NRING), s, zeros_v)
        fire(jnp.int32(c_tail), s)

    for s in range(NRING):
        wait_slot(s)


@jax.jit
def _onehot_sc(x_flat, zeros_tile):
    mesh = plsc.VectorSubcoreMesh(core_axis_name="c", subcore_axis_name="s")
    kern = pl.kernel(
        _body,
        out_type=jax.ShapeDtypeStruct((TPW, NW, 8, 128), jnp.float32),
        mesh=mesh,
        compiler_params=pltpu.CompilerParams(needs_layout_passes=False),
        scratch_types=(
            [pltpu.VMEM((128 * NJ,), jnp.int32),
             pltpu.VMEM((NJ, 128), jnp.int32)]
            + [pltpu.VMEM((NT, 8, 128), jnp.float32) for _ in range(NRING)]
            + [pltpu.SemaphoreType.DMA for _ in range(NRING)]
        ),
    )
    return kern(x_flat, zeros_tile)


def kernel(x):
    x_flat = x.reshape(NI * NJ).astype(jnp.int32)
    zeros_tile = jnp.zeros((NT, 8, 128), jnp.float32)
    out = _onehot_sc(x_flat, zeros_tile)
    o5 = out.reshape(NJ, KT, NW, 8, 128)
    return o5.transpose(2, 4, 0, 1, 3).reshape(NI, NJ, NUM_CL)
